# Initial kernel scaffold; baseline (speedup 1.0000x reference)
import jax
import jax.numpy as jnp
from jax.experimental import pallas as pl


def kernel(x, edge_index, batch, W1, b1, W2, b2):
    raise NotImplementedError("write your pallas kernel here")


# speedup vs baseline: 17.4701x; 17.4701x over previous
import jax
import jax.numpy as jnp
from jax import lax
from jax.experimental import pallas as pl
from jax.experimental.pallas import tpu as pltpu
from jax.experimental.pallas import tpu_sc as plsc

N_NODES = 10000
N_EDGES = 320000
F_DIM = 128
N_GRAPHS = 64

NC = 2
NS = 16
NW = NC * NS
LANES = 16
EPT = N_EDGES // NW
CHUNK = 80
NCHUNKS = EPT // CHUNK
N_PAD = 10112


def _mesh():
    return plsc.VectorSubcoreMesh(core_axis_name="c", subcore_axis_name="s")


def _deg_body(dst_hbm, zeros_hbm, out_hbm, didx_all, ones_v, deg_sh):
    cid = lax.axis_index("c")
    sid = lax.axis_index("s")
    wid = sid * NC + cid

    one = jnp.full((LANES,), 1.0, jnp.float32)
    for r in range(CHUNK):
        for j in range(F_DIM // LANES):
            ones_v[r, pl.ds(j * LANES, LANES)] = one

    pltpu.sync_copy(dst_hbm.at[wid], didx_all)

    @pl.when(sid == 0)
    def _():
        pltpu.sync_copy(zeros_hbm, deg_sh)

    plsc.subcore_barrier()

    @pl.loop(0, NCHUNKS)
    def _(c):
        pltpu.sync_copy(ones_v, deg_sh.at[didx_all.at[c]], add=True)

    plsc.subcore_barrier()

    @pl.when(sid == 0)
    def _():
        pltpu.sync_copy(deg_sh, out_hbm.at[cid])


_deg_call = pl.kernel(
    _deg_body,
    out_type=jax.ShapeDtypeStruct((NC, N_PAD, F_DIM), jnp.float32),
    mesh=_mesh(),
    scratch_types=[
        pltpu.VMEM((NCHUNKS, CHUNK), jnp.int32),
        pltpu.VMEM((CHUNK, F_DIM), jnp.float32),
        pltpu.VMEM_SHARED((N_PAD, F_DIM), jnp.float32),
    ],
)


def _agg_body(y_hbm, src_hbm, dst_hbm, zeros_hbm, out_hbm,
              sidx_all, didx_all, rows_v, acc_sh, sem):
    cid = lax.axis_index("c")
    sid = lax.axis_index("s")
    wid = sid * NC + cid

    pltpu.sync_copy(src_hbm.at[wid], sidx_all)
    pltpu.sync_copy(dst_hbm.at[wid], didx_all)

    @pl.when(sid == 0)
    def _():
        pltpu.sync_copy(zeros_hbm, acc_sh)

    plsc.subcore_barrier()

    @pl.loop(0, NCHUNKS)
    def _(c):
        pltpu.async_copy(y_hbm.at[sidx_all.at[c]], rows_v, sem).wait()
        pltpu.sync_copy(rows_v, acc_sh.at[didx_all.at[c]], add=True)

    plsc.subcore_barrier()

    @pl.when(sid == 0)
    def _():
        pltpu.sync_copy(acc_sh, out_hbm.at[cid])


_agg_call = pl.kernel(
    _agg_body,
    out_type=jax.ShapeDtypeStruct((NC, N_PAD, F_DIM), jnp.float32),
    mesh=_mesh(),
    scratch_types=[
        pltpu.VMEM((NCHUNKS, CHUNK), jnp.int32),
        pltpu.VMEM((NCHUNKS, CHUNK), jnp.int32),
        pltpu.VMEM((CHUNK, F_DIM), jnp.float32),
        pltpu.VMEM_SHARED((N_PAD, F_DIM), jnp.float32),
        pltpu.SemaphoreType.DMA,
    ],
)


def _tc1_body(x_ref, w_ref, degp_ref, y_ref, dis_ref):
    deg = degp_ref[0, :, 0:1] + degp_ref[1, :, 0:1] + 1.0
    dis = lax.rsqrt(deg)
    xw = jnp.dot(x_ref[:], w_ref[:], preferred_element_type=jnp.float32)
    y_ref[:] = xw * dis
    dis_ref[:] = dis


def _tc2_body(parts_ref, y_ref, dis_ref, w_ref, b_ref, y2_ref):
    agg = parts_ref[0] + parts_ref[1] + y_ref[:]
    h = jnp.maximum(dis_ref[:] * agg + b_ref[:], 0.0)
    y2_ref[:] = (
        jnp.dot(h, w_ref[:], preferred_element_type=jnp.float32) * dis_ref[:]
    )


def _tc3_body(parts_ref, y2_ref, dis_ref, b_ref, batch_ref, out_ref):
    z = dis_ref[:] * (parts_ref[0] + parts_ref[1] + y2_ref[:]) + b_ref[:]
    gids = lax.broadcasted_iota(jnp.int32, (1, N_GRAPHS), 1)
    p = (batch_ref[:] == gids).astype(jnp.float32)
    sums = lax.dot_general(
        p, z, (((0,), (0,)), ((), ())), preferred_element_type=jnp.float32
    )
    ones_col = jnp.ones((z.shape[0], 1), jnp.float32)
    counts = lax.dot_general(
        p, ones_col, (((0,), (0,)), ((), ())),
        preferred_element_type=jnp.float32,
    )
    out_ref[:] = sums / jnp.maximum(counts, 1.0)


_tc1 = pl.pallas_call(
    _tc1_body,
    out_shape=(
        jax.ShapeDtypeStruct((N_NODES, F_DIM), jnp.float32),
        jax.ShapeDtypeStruct((N_NODES, 1), jnp.float32),
    ),
)
_tc2 = pl.pallas_call(
    _tc2_body, out_shape=jax.ShapeDtypeStruct((N_NODES, F_DIM), jnp.float32)
)
_tc3 = pl.pallas_call(
    _tc3_body, out_shape=jax.ShapeDtypeStruct((N_GRAPHS, F_DIM), jnp.float32)
)


def kernel(x, edge_index, batch, W1, b1, W2, b2):
    src = edge_index[0].reshape(NW, NCHUNKS, CHUNK)
    dst = edge_index[1].reshape(NW, NCHUNKS, CHUNK)
    zeros = jnp.zeros((N_PAD, F_DIM), jnp.float32)
    b1r = b1.reshape(1, F_DIM)
    b2r = b2.reshape(1, F_DIM)
    batch2 = batch.reshape(N_NODES, 1)

    degp = _deg_call(dst, zeros)[:, :N_NODES]
    y1, dis = _tc1(x, W1, degp)
    parts1 = _agg_call(y1, src, dst, zeros)[:, :N_NODES]
    y2 = _tc2(parts1, y1, dis, W2, b1r)
    parts2 = _agg_call(y2, src, dst, zeros)[:, :N_NODES]
    out = _tc3(parts2, y2, dis, b2r, batch2)
    return out


# speedup vs baseline: 19.5678x; 1.1201x over previous
import jax
import jax.numpy as jnp
from jax import lax
from jax.experimental import pallas as pl
from jax.experimental.pallas import tpu as pltpu
from jax.experimental.pallas import tpu_sc as plsc

N_NODES = 10000
N_EDGES = 320000
F_DIM = 128
N_GRAPHS = 64

NC = 2
NS = 16
NW = NC * NS
LANES = 16
EPT = N_EDGES // NW
CHUNK = 125
NCHUNKS = EPT // CHUNK
N_PAD = 10112


def _mesh():
    return plsc.VectorSubcoreMesh(core_axis_name="c", subcore_axis_name="s")


def _deg_body(dst_hbm, zeros_hbm, out_hbm, didx_all, ones_v, deg_sh):
    cid = lax.axis_index("c")
    sid = lax.axis_index("s")
    wid = sid * NC + cid

    one = jnp.full((LANES,), 1.0, jnp.float32)
    for r in range(CHUNK):
        for j in range(F_DIM // LANES):
            ones_v[r, pl.ds(j * LANES, LANES)] = one

    pltpu.sync_copy(dst_hbm.at[wid], didx_all)

    @pl.when(sid == 0)
    def _():
        pltpu.sync_copy(zeros_hbm, deg_sh)

    plsc.subcore_barrier()

    @pl.loop(0, NCHUNKS)
    def _(c):
        pltpu.sync_copy(ones_v, deg_sh.at[didx_all.at[c]], add=True)

    plsc.subcore_barrier()

    @pl.when(sid == 0)
    def _():
        pltpu.sync_copy(deg_sh, out_hbm.at[cid])


_deg_call = pl.kernel(
    _deg_body,
    out_type=jax.ShapeDtypeStruct((NC, N_PAD, F_DIM), jnp.float32),
    mesh=_mesh(),
    scratch_types=[
        pltpu.VMEM((NCHUNKS, CHUNK), jnp.int32),
        pltpu.VMEM((CHUNK, F_DIM), jnp.float32),
        pltpu.VMEM_SHARED((N_PAD, F_DIM), jnp.float32),
    ],
)


def _agg_body(y_hbm, src_hbm, dst_hbm, zeros_hbm, out_hbm,
              sidx_all, didx_all, rows_v, acc_sh, sem):
    cid = lax.axis_index("c")
    sid = lax.axis_index("s")
    wid = sid * NC + cid

    pltpu.sync_copy(src_hbm.at[wid], sidx_all)
    pltpu.sync_copy(dst_hbm.at[wid], didx_all)

    @pl.when(sid == 0)
    def _():
        pltpu.sync_copy(zeros_hbm, acc_sh)

    plsc.subcore_barrier()

    @pl.loop(0, NCHUNKS)
    def _(c):
        pltpu.async_copy(y_hbm.at[sidx_all.at[c]], rows_v, sem).wait()
        pltpu.sync_copy(rows_v, acc_sh.at[didx_all.at[c]], add=True)

    plsc.subcore_barrier()

    @pl.when(sid == 0)
    def _():
        pltpu.sync_copy(acc_sh, out_hbm.at[cid])


_agg_call = pl.kernel(
    _agg_body,
    out_type=jax.ShapeDtypeStruct((NC, N_PAD, F_DIM), jnp.float32),
    mesh=_mesh(),
    scratch_types=[
        pltpu.VMEM((NCHUNKS, CHUNK), jnp.int32),
        pltpu.VMEM((NCHUNKS, CHUNK), jnp.int32),
        pltpu.VMEM((CHUNK, F_DIM), jnp.float32),
        pltpu.VMEM_SHARED((N_PAD, F_DIM), jnp.float32),
        pltpu.SemaphoreType.DMA,
    ],
)


def _tc1_body(x_ref, w_ref, degp_ref, y_ref, dis_ref):
    deg = degp_ref[0, :, 0:1] + degp_ref[1, :, 0:1] + 1.0
    dis = lax.rsqrt(deg)
    xw = jnp.dot(x_ref[:], w_ref[:], preferred_element_type=jnp.float32)
    y_ref[:] = xw * dis
    dis_ref[:] = dis


def _tc2_body(parts_ref, y_ref, dis_ref, w_ref, b_ref, y2_ref):
    agg = parts_ref[0] + parts_ref[1] + y_ref[:]
    h = jnp.maximum(dis_ref[:] * agg + b_ref[:], 0.0)
    y2_ref[:] = (
        jnp.dot(h, w_ref[:], preferred_element_type=jnp.float32) * dis_ref[:]
    )


def _tc3_body(parts_ref, y2_ref, dis_ref, b_ref, batch_ref, out_ref):
    z = dis_ref[:] * (parts_ref[0] + parts_ref[1] + y2_ref[:]) + b_ref[:]
    gids = lax.broadcasted_iota(jnp.int32, (1, N_GRAPHS), 1)
    p = (batch_ref[:] == gids).astype(jnp.float32)
    sums = lax.dot_general(
        p, z, (((0,), (0,)), ((), ())), preferred_element_type=jnp.float32
    )
    ones_col = jnp.ones((z.shape[0], 1), jnp.float32)
    counts = lax.dot_general(
        p, ones_col, (((0,), (0,)), ((), ())),
        preferred_element_type=jnp.float32,
    )
    out_ref[:] = sums / jnp.maximum(counts, 1.0)


_tc1 = pl.pallas_call(
    _tc1_body,
    out_shape=(
        jax.ShapeDtypeStruct((N_NODES, F_DIM), jnp.float32),
        jax.ShapeDtypeStruct((N_NODES, 1), jnp.float32),
    ),
)
_tc2 = pl.pallas_call(
    _tc2_body, out_shape=jax.ShapeDtypeStruct((N_NODES, F_DIM), jnp.float32)
)
_tc3 = pl.pallas_call(
    _tc3_body, out_shape=jax.ShapeDtypeStruct((N_GRAPHS, F_DIM), jnp.float32)
)


def kernel(x, edge_index, batch, W1, b1, W2, b2):
    src = edge_index[0].reshape(NW, NCHUNKS, CHUNK)
    dst = edge_index[1].reshape(NW, NCHUNKS, CHUNK)
    zeros = jnp.zeros((N_PAD, F_DIM), jnp.float32)
    b1r = b1.reshape(1, F_DIM)
    b2r = b2.reshape(1, F_DIM)
    batch2 = batch.reshape(N_NODES, 1)

    degp = _deg_call(dst, zeros)[:, :N_NODES]
    y1, dis = _tc1(x, W1, degp)
    parts1 = _agg_call(y1, src, dst, zeros)[:, :N_NODES]
    y2 = _tc2(parts1, y1, dis, W2, b1r)
    parts2 = _agg_call(y2, src, dst, zeros)[:, :N_NODES]
    out = _tc3(parts2, y2, dis, b2r, batch2)
    return out
